# Initial kernel scaffold; baseline (speedup 1.0000x reference)
import jax
import jax.numpy as jnp
from jax.experimental import pallas as pl


def kernel(x, table):
    raise NotImplementedError("write your pallas kernel here")


# speedup vs baseline: 4.4500x; 4.4500x over previous
import functools

import jax
import jax.numpy as jnp
from jax import lax
from jax.experimental import pallas as pl
from jax.experimental.pallas import tpu as pltpu
from jax.experimental.pallas import tpu_sc as plsc


_WINDOW = 128
_LANES = 128


def kernel(x, table):
    B, S = x.shape
    V, E = table.shape
    n = B * S
    idx = x.reshape(1, n)
    tab_p = jnp.pad(table, ((0, 0), (0, _LANES - E)))

    mesh = plsc.VectorSubcoreMesh(core_axis_name="core",
                                  subcore_axis_name="subcore")

    @functools.partial(
        pl.kernel,
        out_type=jax.ShapeDtypeStruct((n * E,), table.dtype),
        mesh=mesh,
        scratch_types=[
            pltpu.VMEM_SHARED((V, _LANES), jnp.float32),
            pltpu.VMEM((_WINDOW, _LANES), jnp.float32),
        ],
    )
    def gather_kernel(tab_hbm, i_hbm, o_hbm, tab_shared, gbuf):
        sid = lax.axis_index("subcore")

        @pl.when(sid == 0)
        def _stage_table():
            pltpu.sync_copy(tab_hbm, tab_shared)

        plsc.subcore_barrier()

        def body(i_vmem, o_vmem):
            pltpu.sync_copy(tab_shared.at[i_vmem.at[0]], gbuf)

            @pl.loop(0, _WINDOW)
            def _pack(r):
                for c in range(E // 16):
                    o_vmem.at[pl.ds(r * E + c * 16, 16)][...] = (
                        gbuf.at[r, pl.ds(c * 16, 16)][...])

        pltpu.emit_pipeline(
            body,
            grid=(n // _WINDOW,),
            in_specs=[pl.BlockSpec((1, _WINDOW), index_map=lambda i: (0, i))],
            out_specs=[pl.BlockSpec((_WINDOW * E,), index_map=lambda i: (i,))],
            core_axis_name=("core", "subcore"),
            dimension_semantics=(pltpu.PARALLEL,),
        )(i_hbm, o_hbm)

    out = gather_kernel(tab_p, idx)
    return out.reshape(B, S, E)


# speedup vs baseline: 4.6365x; 1.0419x over previous
import functools

import jax
import jax.numpy as jnp
from jax import lax
from jax.experimental import pallas as pl
from jax.experimental.pallas import tpu as pltpu
from jax.experimental.pallas import tpu_sc as plsc


_LANES = 128


def kernel(x, table):
    B, S = x.shape
    V, E = table.shape
    idx = x.reshape(B, 1, S)
    tab_p = jnp.pad(table, ((0, 0), (0, _LANES - E)))
    w0 = _LANES
    w1 = S - _LANES

    mesh = plsc.VectorSubcoreMesh(core_axis_name="core",
                                  subcore_axis_name="subcore")

    @functools.partial(
        pl.kernel,
        out_type=jax.ShapeDtypeStruct((B, S, E), table.dtype),
        mesh=mesh,
        scratch_types=[
            pltpu.VMEM_SHARED((V, _LANES), jnp.float32),
            pltpu.VMEM((S, _LANES), jnp.float32),
        ],
    )
    def gather_kernel(tab_hbm, i_hbm, o_hbm, tab_shared, gbuf):
        sid = lax.axis_index("subcore")

        @pl.when(sid == 0)
        def _stage_table():
            pltpu.sync_copy(tab_hbm, tab_shared)

        plsc.subcore_barrier()

        def body(i_vmem, o_vmem):
            pltpu.sync_copy(tab_shared.at[i_vmem.at[0, 0, pl.ds(0, w0)]],
                            gbuf.at[pl.ds(0, w0)])
            pltpu.sync_copy(tab_shared.at[i_vmem.at[0, 0, pl.ds(w0, w1)]],
                            gbuf.at[pl.ds(w0, w1)])

            @pl.loop(0, S)
            def _pack(r):
                for c in range(E // 16):
                    o_vmem.at[0, r, pl.ds(c * 16, 16)][...] = (
                        gbuf.at[r, pl.ds(c * 16, 16)][...])

        pltpu.emit_pipeline(
            body,
            grid=(B,),
            in_specs=[pl.BlockSpec((1, 1, S), index_map=lambda i: (i, 0, 0))],
            out_specs=[pl.BlockSpec((1, S, E), index_map=lambda i: (i, 0, 0))],
            core_axis_name=("core", "subcore"),
            dimension_semantics=(pltpu.PARALLEL,),
        )(i_hbm, o_hbm)

    return gather_kernel(tab_p, idx)


# speedup vs baseline: 9.4535x; 2.0389x over previous
import functools

import jax
import jax.numpy as jnp
from jax import lax
from jax.experimental import pallas as pl
from jax.experimental.pallas import tpu as pltpu
from jax.experimental.pallas import tpu_sc as plsc


_LANES = 128


def kernel(x, table):
    B, S = x.shape
    V, E = table.shape
    idx = x.reshape(B, 1, S)
    tab_p = jnp.pad(table, ((0, 0), (0, _LANES - E)))
    w0 = _LANES
    w1 = S - _LANES

    mesh = plsc.VectorSubcoreMesh(core_axis_name="core",
                                  subcore_axis_name="subcore")

    @functools.partial(
        pl.kernel,
        out_type=jax.ShapeDtypeStruct((B, S, _LANES), table.dtype),
        mesh=mesh,
        scratch_types=[
            pltpu.VMEM_SHARED((V, _LANES), jnp.float32),
        ],
    )
    def gather_kernel(tab_hbm, i_hbm, o_hbm, tab_shared):
        sid = lax.axis_index("subcore")

        @pl.when(sid == 0)
        def _stage_table():
            pltpu.sync_copy(tab_hbm, tab_shared)

        plsc.subcore_barrier()

        def body(i_vmem, o_vmem):
            pltpu.sync_copy(tab_shared.at[i_vmem.at[0, 0, pl.ds(0, w0)]],
                            o_vmem.at[0, pl.ds(0, w0)])
            pltpu.sync_copy(tab_shared.at[i_vmem.at[0, 0, pl.ds(w0, w1)]],
                            o_vmem.at[0, pl.ds(w0, w1)])

        pltpu.emit_pipeline(
            body,
            grid=(B,),
            in_specs=[pl.BlockSpec((1, 1, S), index_map=lambda i: (i, 0, 0))],
            out_specs=[pl.BlockSpec((1, S, _LANES),
                                    index_map=lambda i: (i, 0, 0))],
            core_axis_name=("core", "subcore"),
            dimension_semantics=(pltpu.PARALLEL,),
        )(i_hbm, o_hbm)

    return gather_kernel(tab_p, idx)[:, :, :E]
